# baseline (speedup 1.0000x reference)
import functools
import math

import jax
import jax.numpy as jnp
from jax import lax
from jax.experimental import pallas as pl
from jax.experimental.pallas import tpu as pltpu
from jax.experimental.pallas import tpu_sc as plsc

NUM_SAMPLED = 1024
NUM_CLASSES = 100000
LAMB = 0.1
BATCH = 4096
DIM = 128

_NW = 32
_LOGNC1 = math.log(NUM_CLASSES + 1.0)


def _sc_gather(table, bias, ids, n):
    mesh = plsc.VectorSubcoreMesh(core_axis_name="c", subcore_axis_name="s")
    nb = n // _NW

    @functools.partial(
        pl.kernel,
        out_type=(
            jax.ShapeDtypeStruct((n, DIM), jnp.float32),
            jax.ShapeDtypeStruct((n,), jnp.float32),
        ),
        mesh=mesh,
        scratch_types=[
            pltpu.VMEM((nb,), jnp.int32),
            pltpu.VMEM((nb, DIM), jnp.float32),
            pltpu.VMEM((nb,), jnp.float32),
        ] + [pltpu.SemaphoreType.DMA] * 4,
    )
    def k(table_h, bias_h, ids_h, w_h, b_h, idx, wv, bv, g1, g2, w1, w2):
        wid = lax.axis_index("s") * 2 + lax.axis_index("c")
        base = wid * nb
        pltpu.sync_copy(ids_h.at[pl.ds(base, nb)], idx)
        c1 = pltpu.async_copy(table_h.at[idx], wv, g1)
        c2 = pltpu.async_copy(bias_h.at[idx], bv, g2)
        c1.wait()
        o1 = pltpu.async_copy(wv, w_h.at[pl.ds(base, nb)], w1)
        c2.wait()
        o2 = pltpu.async_copy(bv, b_h.at[pl.ds(base, nb)], w2)
        o1.wait()
        o2.wait()

    return k(table, bias, ids)


_BB = 512


def _tc_sampled_body(xin_ref, sw_ref, sb_ref, nlq_ref, samp_ref, lab_ref,
                     ms_ref, se_ref):
    x = xin_ref[...]
    s_log = lax.dot_general(x, sw_ref[...], (((1,), (1,)), ((), ())),
                            preferred_element_type=jnp.float32)
    s_log = s_log + (sb_ref[...] + nlq_ref[...])
    hit = lab_ref[...] == samp_ref[...]
    s_log = jnp.where(hit, s_log - 1e9, s_log)
    m = jnp.max(s_log, axis=1, keepdims=True)
    ms_ref[...] = m
    se_ref[...] = jnp.sum(jnp.exp(s_log - m), axis=1, keepdims=True)


def _tc_sampled(inputs_x, sw, sb_row, nlq_row, samp_row, lab_col):
    grid = BATCH // _BB
    return pl.pallas_call(
        _tc_sampled_body,
        grid=(grid,),
        in_specs=[
            pl.BlockSpec((_BB, DIM), lambda i: (i, 0)),
            pl.BlockSpec((NUM_SAMPLED, DIM), lambda i: (0, 0)),
            pl.BlockSpec((1, NUM_SAMPLED), lambda i: (0, 0)),
            pl.BlockSpec((1, NUM_SAMPLED), lambda i: (0, 0)),
            pl.BlockSpec((1, NUM_SAMPLED), lambda i: (0, 0)),
            pl.BlockSpec((_BB, 1), lambda i: (i, 0)),
        ],
        out_specs=[
            pl.BlockSpec((_BB, 1), lambda i: (i, 0)),
            pl.BlockSpec((_BB, 1), lambda i: (i, 0)),
        ],
        out_shape=[
            jax.ShapeDtypeStruct((BATCH, 1), jnp.float32),
            jax.ShapeDtypeStruct((BATCH, 1), jnp.float32),
        ],
    )(inputs_x, sw, sb_row, nlq_row, samp_row, lab_col)


def _tc_combine_body(xin_ref, tw_ref, tb_ref, lab_ref, ms_ref, se_ref,
                     out_ref):
    x = xin_ref[...]
    labf = lab_ref[...].astype(jnp.float32)
    tp = (jnp.log(labf + 2.0) - jnp.log(labf + 1.0)) / _LOGNC1
    tq = 1.0 - jnp.exp(NUM_SAMPLED * jnp.log(1.0 - tp))
    t_log = (jnp.sum(x * tw_ref[...], axis=1, keepdims=True)
             + tb_ref[...] - jnp.log(tq + 1e-20))
    ms = ms_ref[...]
    m = jnp.maximum(ms, t_log)
    se = se_ref[...] * jnp.exp(ms - m) + jnp.exp(t_log - m)
    loss = jnp.log(se) + m - t_log
    out_ref[0, 0] = jnp.sum(loss) * jnp.float32(LAMB / BATCH)


def _tc_combine(inputs_x, tw, tb_col, lab_col, ms, se):
    return pl.pallas_call(
        _tc_combine_body,
        grid=(1,),
        in_specs=[
            pl.BlockSpec((BATCH, DIM), lambda i: (0, 0)),
            pl.BlockSpec((BATCH, DIM), lambda i: (0, 0)),
            pl.BlockSpec((BATCH, 1), lambda i: (0, 0)),
            pl.BlockSpec((BATCH, 1), lambda i: (0, 0)),
            pl.BlockSpec((BATCH, 1), lambda i: (0, 0)),
            pl.BlockSpec((BATCH, 1), lambda i: (0, 0)),
        ],
        out_specs=pl.BlockSpec(memory_space=pltpu.SMEM),
        out_shape=jax.ShapeDtypeStruct((1, 1), jnp.float32),
    )(inputs_x, tw, tb_col, lab_col, ms, se)


def _sampled_ids():
    key = jax.random.fold_in(jax.random.key(42), 5)
    u = jax.random.uniform(key, (NUM_SAMPLED,))
    s = jnp.floor(jnp.exp(u * jnp.log(NUM_CLASSES + 1.0))) - 1.0
    return jnp.clip(s, 0, NUM_CLASSES - 1).astype(jnp.int32)


def kernel(inputs_in, labels_in, kernel, bias):
    labels = labels_in[:, 3].astype(jnp.int32)
    sampled = _sampled_ids()
    samp_f = sampled.astype(jnp.float32)
    sp = (jnp.log(samp_f + 2.0) - jnp.log(samp_f + 1.0)) / _LOGNC1
    sq = 1.0 - jnp.exp(NUM_SAMPLED * jnp.log(1.0 - sp))
    nlq_row = (-jnp.log(sq + 1e-20)).reshape(1, NUM_SAMPLED)
    samp_row = sampled.reshape(1, NUM_SAMPLED)

    x = inputs_in[:, 2, :].astype(jnp.float32)
    lab_col = labels.reshape(BATCH, 1)

    sw, sb = _sc_gather(kernel, bias, sampled, NUM_SAMPLED)
    ms, se = _tc_sampled(x, sw, sb.reshape(1, NUM_SAMPLED), nlq_row,
                         samp_row, lab_col)
    tw, tb = _sc_gather(kernel, bias, labels, BATCH)
    out = _tc_combine(x, tw, tb.reshape(BATCH, 1), lab_col, ms, se)
    return out[0, 0]
